# baseline (speedup 1.0000x reference)
import jax
import jax.numpy as jnp
from jax import lax
from jax.experimental import pallas as pl
from jax.experimental.pallas import tpu as pltpu
from jax.experimental.pallas import tpu_sc as plsc

N = 10000
E = 320000
D = 128

NC = 2
NS = 16
NW = NC * NS
CHUNK = 128
PART_CHUNKS = (260, 620, 620, 620, 380)
PART_STARTS = (0, 260, 880, 1500, 2120)
P = len(PART_CHUNKS)
RING = 3
SRING = 2
NPAD = 10240
ROWS_PER_TILE = NPAD // NS
BE = 2560
BN = 2000

_MESH = plsc.VectorSubcoreMesh(core_axis_name="c", subcore_axis_name="s")


def _wid():
    return lax.axis_index("s") * NC + lax.axis_index("c")


def _make_gather(part):
    nch = PART_CHUNKS[part]
    pbase = PART_STARTS[part]
    cpw = nch // NW
    left = nch - cpw * NW
    rounds = cpw // RING
    rem = cpw - rounds * RING
    epart = nch * CHUNK

    def body(nf_hbm, ei_hbm, ai_hbm, aj_hbm,
             idx_s, idx_d, rs0, rs1, rs2, rd0, rd1, rd2,
             sg0, sg1, sg2, sw0, sw1, sw2):
        wid = _wid()
        gbase0 = (pbase + wid * cpw) * CHUNK
        lbase0 = wid * cpw * CHUNK
        rs = (rs0, rs1, rs2)
        rd = (rd0, rd1, rd2)
        sg = (sg0, sg1, sg2)
        sw = (sw0, sw1, sw2)

        pltpu.sync_copy(ei_hbm.at[pl.ds(gbase0, cpw * CHUNK)], idx_s)
        pltpu.sync_copy(ei_hbm.at[pl.ds(E + gbase0, cpw * CHUNK)], idx_d)

        def issue_gather(c, b):
            off = c * CHUNK
            pltpu.async_copy(nf_hbm.at[idx_s.at[pl.ds(off, CHUNK)]],
                             rs[b], sg[b])
            pltpu.async_copy(nf_hbm.at[idx_d.at[pl.ds(off, CHUNK)]],
                             rd[b], sg[b])

        def wait_gather(b):
            pltpu.make_async_copy(nf_hbm.at[pl.ds(0, CHUNK)],
                                  rs[b], sg[b]).wait()
            pltpu.make_async_copy(nf_hbm.at[pl.ds(0, CHUNK)],
                                  rd[b], sg[b]).wait()

        def issue_wb(c, b):
            base = lbase0 + c * CHUNK
            pltpu.async_copy(rs[b], ai_hbm.at[pl.ds(base, CHUNK)], sw[b])
            pltpu.async_copy(rd[b], aj_hbm.at[pl.ds(base, CHUNK)], sw[b])

        def wait_wb(b):
            pltpu.make_async_copy(rs[b], ai_hbm.at[pl.ds(0, CHUNK)],
                                  sw[b]).wait()
            pltpu.make_async_copy(rd[b], aj_hbm.at[pl.ds(0, CHUNK)],
                                  sw[b]).wait()

        for b in range(RING):
            issue_gather(b, b)

        def step(k, carry):
            for b in range(RING):
                c = k * RING + b
                wait_gather(b)
                issue_wb(c, b)
                wait_wb(b)

                @pl.when(k < rounds - 1)
                def _():
                    issue_gather(c + RING, b)
            return carry

        lax.fori_loop(0, rounds, step, 0)

        for c in range(rounds * RING, cpw):
            b = c % RING
            issue_gather(c, b)
            wait_gather(b)
            issue_wb(c, b)
            wait_wb(b)

        @pl.when(wid < left)
        def _():
            lc = NW * cpw + wid
            gb = (pbase + lc) * CHUNK
            lb = lc * CHUNK
            pltpu.sync_copy(ei_hbm.at[pl.ds(gb, CHUNK)],
                            idx_s.at[pl.ds(0, CHUNK)])
            pltpu.sync_copy(ei_hbm.at[pl.ds(E + gb, CHUNK)],
                            idx_d.at[pl.ds(0, CHUNK)])
            pltpu.async_copy(nf_hbm.at[idx_s.at[pl.ds(0, CHUNK)]], rs0, sg0)
            pltpu.async_copy(nf_hbm.at[idx_d.at[pl.ds(0, CHUNK)]], rd0, sg0)
            wait_gather(0)
            pltpu.sync_copy(rs0, ai_hbm.at[pl.ds(lb, CHUNK)])
            pltpu.sync_copy(rd0, aj_hbm.at[pl.ds(lb, CHUNK)])

    return pl.kernel(
        body,
        out_type=(jax.ShapeDtypeStruct((epart, D), jnp.float32),
                  jax.ShapeDtypeStruct((epart, D), jnp.float32)),
        mesh=_MESH,
        scratch_types=(
            [pltpu.VMEM((cpw * CHUNK,), jnp.int32)] * 2
            + [pltpu.VMEM((CHUNK, D), jnp.float32)] * 6
            + [pltpu.SemaphoreType.DMA] * 6
        ),
    )


def _make_scatter(parts):
    nmsg = len(parts)
    max_cpw = max(PART_CHUNKS[p] // NW for p in parts)

    def body(*refs):
        msgs = refs[:nmsg]
        dst3_hbm, zero_hbm, out_hbm = refs[nmsg:nmsg + 3]
        idx_v, m0, m1, agg_sh, sl0, sl1, ss0, ss1 = refs[nmsg + 3:]
        cid = lax.axis_index("c")
        sid = lax.axis_index("s")
        r0 = sid * ROWS_PER_TILE
        pltpu.sync_copy(zero_hbm.at[pl.ds(r0, ROWS_PER_TILE)],
                        agg_sh.at[pl.ds(r0, ROWS_PER_TILE)])
        plsc.subcore_barrier()

        wid = sid * NC + cid
        m = (m0, m1)
        sl = (sl0, sl1)
        ss = (ss0, ss1)

        def run_phase(msg_hbm, part):
            nch = PART_CHUNKS[part]
            pbase = PART_STARTS[part]
            cpw = nch // NW
            left = nch - cpw * NW
            nrounds = cpw // SRING

            pltpu.sync_copy(dst3_hbm.at[pl.ds(pbase + wid * cpw, cpw)],
                            idx_v.at[pl.ds(0, cpw)])

            def issue_load(c, b):
                base = (wid * cpw + c) * CHUNK
                pltpu.async_copy(msg_hbm.at[pl.ds(base, CHUNK)], m[b], sl[b])

            def wait_load(b):
                pltpu.make_async_copy(msg_hbm.at[pl.ds(0, CHUNK)],
                                      m[b], sl[b]).wait()

            def issue_scatter(c, b):
                pltpu.async_copy(m[b], agg_sh.at[idx_v.at[c, 0]], ss[b],
                                 add=True)

            def wait_scatter(b):
                pltpu.make_async_copy(m[b], agg_sh.at[pl.ds(0, CHUNK)],
                                      ss[b]).wait()

            for b in range(SRING):
                issue_load(b, b)

            def step(k, carry):
                for b in range(SRING):
                    c = k * SRING + b
                    wait_load(b)
                    issue_scatter(c, b)
                    wait_scatter(b)

                    @pl.when(k < nrounds - 1)
                    def _():
                        issue_load(c + SRING, b)
                return carry

            lax.fori_loop(0, nrounds, step, 0)

            for c in range(nrounds * SRING, cpw):
                pltpu.sync_copy(
                    msg_hbm.at[pl.ds((wid * cpw + c) * CHUNK, CHUNK)], m0)
                pltpu.sync_copy(m0, agg_sh.at[idx_v.at[c, 0]], add=True)

            @pl.when(wid < left)
            def _():
                lc = NW * cpw + wid
                pltpu.sync_copy(dst3_hbm.at[pl.ds(pbase + lc, 1)],
                                idx_v.at[pl.ds(0, 1)])
                pltpu.sync_copy(msg_hbm.at[pl.ds(lc * CHUNK, CHUNK)], m0)
                pltpu.sync_copy(m0, agg_sh.at[idx_v.at[0, 0]], add=True)

        for i, part in enumerate(parts):
            run_phase(msgs[i], part)

        plsc.subcore_barrier()
        last = N - (NS - 1) * ROWS_PER_TILE

        @pl.when(sid < NS - 1)
        def _():
            pltpu.sync_copy(agg_sh.at[pl.ds(r0, ROWS_PER_TILE)],
                            out_hbm.at[cid, pl.ds(r0, ROWS_PER_TILE)])

        @pl.when(sid == NS - 1)
        def _():
            pltpu.sync_copy(agg_sh.at[pl.ds(r0, last)],
                            out_hbm.at[cid, pl.ds(r0, last)])

    return pl.kernel(
        body,
        out_type=jax.ShapeDtypeStruct((NC, N, D), jnp.float32),
        mesh=_MESH,
        scratch_types=(
            [pltpu.VMEM((max_cpw, 1, CHUNK), jnp.int32)]
            + [pltpu.VMEM((CHUNK, D), jnp.float32)] * 2
            + [pltpu.VMEM_SHARED((NPAD, D), jnp.float32)]
            + [pltpu.SemaphoreType.DMA] * 4
        ),
    )


def _main_body(ne_in, ai, aj, ef, sew, snw, wve, wge, wvn, wgn,
               bve, bge, bvn, bgn, ne_out, msg_out):
    del ne_in
    ai_ = ai[...].astype(jnp.bfloat16)
    aj_ = aj[...].astype(jnp.bfloat16)
    ef_ = ef[...]
    ef_b = ef_.astype(jnp.bfloat16)

    def mm3(x2, w):
        wb = w.astype(jnp.bfloat16)
        return (jnp.dot(ai_, wb[0:D], preferred_element_type=jnp.float32)
                + jnp.dot(x2, wb[D:2 * D], preferred_element_type=jnp.float32)
                + jnp.dot(aj_, wb[2 * D:3 * D],
                          preferred_element_type=jnp.float32))

    core = jax.nn.silu(mm3(ef_b, wve[...]) + bve[...])
    gate = jax.nn.sigmoid(mm3(ef_b, wge[...]) + bge[...])
    ne = ef_ + core * gate * sew[...]
    ne_out[...] = ne
    ne_b = ne.astype(jnp.bfloat16)
    core2 = jax.nn.silu(mm3(ne_b, wvn[...]) + bvn[...])
    gate2 = jax.nn.sigmoid(mm3(ne_b, wgn[...]) + bgn[...])
    msg_out[...] = core2 * gate2 * snw[...]


def _main_body_first(ai, aj, ef, sew, snw, wve, wge, wvn, wgn,
                     bve, bge, bvn, bgn, ne_out, msg_out):
    _main_body(None, ai, aj, ef, sew, snw, wve, wge, wvn, wgn,
               bve, bge, bvn, bgn, ne_out, msg_out)


def _tc_main(part, ne_acc, atom_i, atom_j, ef, sew, snw, wve, wge, wvn, wgn,
             bve, bge, bvn, bgn):
    epart = PART_CHUNKS[part] * CHUNK
    boff = PART_STARTS[part] * CHUNK // BE
    nbe = epart // BE
    first = part == 0
    pb = lambda i: (i, 0)
    gb = lambda i, boff=boff: (i + boff, 0)
    wb = lambda i: (0, 0)
    specs = [
        pl.BlockSpec((BE, D), pb),
        pl.BlockSpec((BE, D), pb),
        pl.BlockSpec((BE, D), gb),
        pl.BlockSpec((BE, D), gb),
        pl.BlockSpec((BE, D), gb),
        pl.BlockSpec((3 * D, D), wb),
        pl.BlockSpec((3 * D, D), wb),
        pl.BlockSpec((3 * D, D), wb),
        pl.BlockSpec((3 * D, D), wb),
        pl.BlockSpec((1, D), wb),
        pl.BlockSpec((1, D), wb),
        pl.BlockSpec((1, D), wb),
        pl.BlockSpec((1, D), wb),
    ]
    if not first:
        specs = [pl.BlockSpec(memory_space=pl.ANY)] + specs
    call = pl.pallas_call(
        _main_body_first if first else _main_body,
        grid=(nbe,),
        in_specs=specs,
        out_specs=[pl.BlockSpec((BE, D), gb), pl.BlockSpec((BE, D), pb)],
        out_shape=[jax.ShapeDtypeStruct((E, D), jnp.float32),
                   jax.ShapeDtypeStruct((epart, D), jnp.float32)],
        input_output_aliases={} if first else {0: 0},
    )
    args = (atom_i, atom_j, ef, sew, snw, wve, wge, wvn, wgn,
            bve, bge, bvn, bgn)
    if not first:
        args = (ne_acc,) + args
    return call(*args)


def _final_body(nf, a0, a1, wout, out):
    a = (a0[0] + a0[1]) + (a1[0] + a1[1])
    out[...] = nf[...] + jnp.dot(a, wout[...],
                                 preferred_element_type=jnp.float32)


def _tc_final(nf, aggs, wout):
    ab = lambda i: (0, i, 0)
    return pl.pallas_call(
        _final_body,
        grid=(N // BN,),
        in_specs=[pl.BlockSpec((BN, D), lambda i: (i, 0))]
        + [pl.BlockSpec((NC, BN, D), ab)] * len(aggs)
        + [pl.BlockSpec((D, D), lambda i: (0, 0))],
        out_specs=pl.BlockSpec((BN, D), lambda i: (i, 0)),
        out_shape=jax.ShapeDtypeStruct((N, D), jnp.float32),
    )(nf, *aggs, wout)


def kernel(node_features, edge_features, state_attr, shared_node_weights,
           shared_edge_weights, Wg_e, bg_e, Wv_e, bv_e,
           Wg_n, bg_n, Wv_n, bv_n, W_out, edge_index):
    ei_flat = edge_index.astype(jnp.int32).reshape(2 * E)
    dst3 = edge_index[1].astype(jnp.int32).reshape(E // CHUNK, 1, CHUNK)
    zeros = jnp.zeros((NPAD, D), jnp.float32)
    biases = (bv_e.reshape(1, D), bg_e.reshape(1, D),
              bv_n.reshape(1, D), bg_n.reshape(1, D))

    ne_full = None
    msgs = []
    for p in range(P):
        atom_i, atom_j = _make_gather(p)(node_features, ei_flat)
        ne_full, msg = _tc_main(
            p, ne_full, atom_i, atom_j, edge_features,
            shared_edge_weights, shared_node_weights,
            Wv_e, Wg_e, Wv_n, Wg_n, *biases)
        msgs.append(msg)

    agg_a = _make_scatter((0, 1, 2))(msgs[0], msgs[1], msgs[2], dst3, zeros)
    agg_b = _make_scatter((3, 4))(msgs[3], msgs[4], dst3, zeros)

    new_node = _tc_final(node_features, [agg_a, agg_b], W_out)
    return new_node, ne_full, state_attr


# speedup vs baseline: 1.0071x; 1.0071x over previous
import jax
import jax.numpy as jnp
from jax import lax
from jax.experimental import pallas as pl
from jax.experimental.pallas import tpu as pltpu
from jax.experimental.pallas import tpu_sc as plsc

N = 10000
E = 320000
D = 128

NC = 2
NS = 16
NW = NC * NS
CHUNK = 128
PART_CHUNKS = (500, 500, 500, 500, 500)
PART_STARTS = (0, 500, 1000, 1500, 2000)
P = len(PART_CHUNKS)
RING = 3
SRING = 2
NPAD = 10240
ROWS_PER_TILE = NPAD // NS
BE = 3200
BN = 2000

_MESH = plsc.VectorSubcoreMesh(core_axis_name="c", subcore_axis_name="s")


def _wid():
    return lax.axis_index("s") * NC + lax.axis_index("c")


def _make_gather(part):
    nch = PART_CHUNKS[part]
    pbase = PART_STARTS[part]
    cpw = nch // NW
    left = nch - cpw * NW
    rounds = cpw // RING
    rem = cpw - rounds * RING
    epart = nch * CHUNK

    def body(nf_hbm, ei_hbm, ai_hbm, aj_hbm,
             idx_s, idx_d, rs0, rs1, rs2, rd0, rd1, rd2,
             sg0, sg1, sg2, sw0, sw1, sw2):
        wid = _wid()
        gbase0 = (pbase + wid * cpw) * CHUNK
        lbase0 = wid * cpw * CHUNK
        rs = (rs0, rs1, rs2)
        rd = (rd0, rd1, rd2)
        sg = (sg0, sg1, sg2)
        sw = (sw0, sw1, sw2)

        pltpu.sync_copy(ei_hbm.at[pl.ds(gbase0, cpw * CHUNK)], idx_s)
        pltpu.sync_copy(ei_hbm.at[pl.ds(E + gbase0, cpw * CHUNK)], idx_d)

        def issue_gather(c, b):
            off = c * CHUNK
            pltpu.async_copy(nf_hbm.at[idx_s.at[pl.ds(off, CHUNK)]],
                             rs[b], sg[b])
            pltpu.async_copy(nf_hbm.at[idx_d.at[pl.ds(off, CHUNK)]],
                             rd[b], sg[b])

        def wait_gather(b):
            pltpu.make_async_copy(nf_hbm.at[pl.ds(0, CHUNK)],
                                  rs[b], sg[b]).wait()
            pltpu.make_async_copy(nf_hbm.at[pl.ds(0, CHUNK)],
                                  rd[b], sg[b]).wait()

        def issue_wb(c, b):
            base = lbase0 + c * CHUNK
            pltpu.async_copy(rs[b], ai_hbm.at[pl.ds(base, CHUNK)], sw[b])
            pltpu.async_copy(rd[b], aj_hbm.at[pl.ds(base, CHUNK)], sw[b])

        def wait_wb(b):
            pltpu.make_async_copy(rs[b], ai_hbm.at[pl.ds(0, CHUNK)],
                                  sw[b]).wait()
            pltpu.make_async_copy(rd[b], aj_hbm.at[pl.ds(0, CHUNK)],
                                  sw[b]).wait()

        for b in range(RING):
            issue_gather(b, b)

        def step(k, carry):
            for b in range(RING):
                c = k * RING + b
                wait_gather(b)
                issue_wb(c, b)
                wait_wb(b)

                @pl.when(k < rounds - 1)
                def _():
                    issue_gather(c + RING, b)
            return carry

        lax.fori_loop(0, rounds, step, 0)

        for c in range(rounds * RING, cpw):
            b = c % RING
            issue_gather(c, b)
            wait_gather(b)
            issue_wb(c, b)
            wait_wb(b)

        @pl.when(wid < left)
        def _():
            lc = NW * cpw + wid
            gb = (pbase + lc) * CHUNK
            lb = lc * CHUNK
            pltpu.sync_copy(ei_hbm.at[pl.ds(gb, CHUNK)],
                            idx_s.at[pl.ds(0, CHUNK)])
            pltpu.sync_copy(ei_hbm.at[pl.ds(E + gb, CHUNK)],
                            idx_d.at[pl.ds(0, CHUNK)])
            pltpu.async_copy(nf_hbm.at[idx_s.at[pl.ds(0, CHUNK)]], rs0, sg0)
            pltpu.async_copy(nf_hbm.at[idx_d.at[pl.ds(0, CHUNK)]], rd0, sg0)
            wait_gather(0)
            pltpu.sync_copy(rs0, ai_hbm.at[pl.ds(lb, CHUNK)])
            pltpu.sync_copy(rd0, aj_hbm.at[pl.ds(lb, CHUNK)])

    return pl.kernel(
        body,
        out_type=(jax.ShapeDtypeStruct((epart, D), jnp.float32),
                  jax.ShapeDtypeStruct((epart, D), jnp.float32)),
        mesh=_MESH,
        scratch_types=(
            [pltpu.VMEM((cpw * CHUNK,), jnp.int32)] * 2
            + [pltpu.VMEM((CHUNK, D), jnp.float32)] * 6
            + [pltpu.SemaphoreType.DMA] * 6
        ),
    )


def _make_scatter(parts):
    nmsg = len(parts)
    max_cpw = max(PART_CHUNKS[p] // NW for p in parts)

    def body(*refs):
        msgs = refs[:nmsg]
        dst3_hbm, zero_hbm, out_hbm = refs[nmsg:nmsg + 3]
        idx_v, m0, m1, agg_sh, sl0, sl1, ss0, ss1 = refs[nmsg + 3:]
        cid = lax.axis_index("c")
        sid = lax.axis_index("s")
        r0 = sid * ROWS_PER_TILE
        pltpu.sync_copy(zero_hbm.at[pl.ds(r0, ROWS_PER_TILE)],
                        agg_sh.at[pl.ds(r0, ROWS_PER_TILE)])
        plsc.subcore_barrier()

        wid = sid * NC + cid
        m = (m0, m1)
        sl = (sl0, sl1)
        ss = (ss0, ss1)

        def run_phase(msg_hbm, part):
            nch = PART_CHUNKS[part]
            pbase = PART_STARTS[part]
            cpw = nch // NW
            left = nch - cpw * NW
            nrounds = cpw // SRING

            pltpu.sync_copy(dst3_hbm.at[pl.ds(pbase + wid * cpw, cpw)],
                            idx_v.at[pl.ds(0, cpw)])

            def issue_load(c, b):
                base = (wid * cpw + c) * CHUNK
                pltpu.async_copy(msg_hbm.at[pl.ds(base, CHUNK)], m[b], sl[b])

            def wait_load(b):
                pltpu.make_async_copy(msg_hbm.at[pl.ds(0, CHUNK)],
                                      m[b], sl[b]).wait()

            def issue_scatter(c, b):
                pltpu.async_copy(m[b], agg_sh.at[idx_v.at[c, 0]], ss[b],
                                 add=True)

            def wait_scatter(b):
                pltpu.make_async_copy(m[b], agg_sh.at[pl.ds(0, CHUNK)],
                                      ss[b]).wait()

            for b in range(SRING):
                issue_load(b, b)

            def step(k, carry):
                for b in range(SRING):
                    c = k * SRING + b
                    wait_load(b)
                    issue_scatter(c, b)
                    wait_scatter(b)

                    @pl.when(k < nrounds - 1)
                    def _():
                        issue_load(c + SRING, b)
                return carry

            lax.fori_loop(0, nrounds, step, 0)

            for c in range(nrounds * SRING, cpw):
                pltpu.sync_copy(
                    msg_hbm.at[pl.ds((wid * cpw + c) * CHUNK, CHUNK)], m0)
                pltpu.sync_copy(m0, agg_sh.at[idx_v.at[c, 0]], add=True)

            @pl.when(wid < left)
            def _():
                lc = NW * cpw + wid
                pltpu.sync_copy(dst3_hbm.at[pl.ds(pbase + lc, 1)],
                                idx_v.at[pl.ds(0, 1)])
                pltpu.sync_copy(msg_hbm.at[pl.ds(lc * CHUNK, CHUNK)], m0)
                pltpu.sync_copy(m0, agg_sh.at[idx_v.at[0, 0]], add=True)

        for i, part in enumerate(parts):
            run_phase(msgs[i], part)

        plsc.subcore_barrier()
        last = N - (NS - 1) * ROWS_PER_TILE

        @pl.when(sid < NS - 1)
        def _():
            pltpu.sync_copy(agg_sh.at[pl.ds(r0, ROWS_PER_TILE)],
                            out_hbm.at[cid, pl.ds(r0, ROWS_PER_TILE)])

        @pl.when(sid == NS - 1)
        def _():
            pltpu.sync_copy(agg_sh.at[pl.ds(r0, last)],
                            out_hbm.at[cid, pl.ds(r0, last)])

    return pl.kernel(
        body,
        out_type=jax.ShapeDtypeStruct((NC, N, D), jnp.float32),
        mesh=_MESH,
        scratch_types=(
            [pltpu.VMEM((max_cpw, 1, CHUNK), jnp.int32)]
            + [pltpu.VMEM((CHUNK, D), jnp.float32)] * 2
            + [pltpu.VMEM_SHARED((NPAD, D), jnp.float32)]
            + [pltpu.SemaphoreType.DMA] * 4
        ),
    )


def _main_body(ne_in, ai, aj, ef, sew, snw, wve, wge, wvn, wgn,
               bve, bge, bvn, bgn, ne_out, msg_out):
    del ne_in
    ai_ = ai[...].astype(jnp.bfloat16)
    aj_ = aj[...].astype(jnp.bfloat16)
    ef_ = ef[...]
    ef_b = ef_.astype(jnp.bfloat16)

    def mm3(x2, w):
        wb = w.astype(jnp.bfloat16)
        return (jnp.dot(ai_, wb[0:D], preferred_element_type=jnp.float32)
                + jnp.dot(x2, wb[D:2 * D], preferred_element_type=jnp.float32)
                + jnp.dot(aj_, wb[2 * D:3 * D],
                          preferred_element_type=jnp.float32))

    core = jax.nn.silu(mm3(ef_b, wve[...]) + bve[...])
    gate = jax.nn.sigmoid(mm3(ef_b, wge[...]) + bge[...])
    ne = ef_ + core * gate * sew[...]
    ne_out[...] = ne
    ne_b = ne.astype(jnp.bfloat16)
    core2 = jax.nn.silu(mm3(ne_b, wvn[...]) + bvn[...])
    gate2 = jax.nn.sigmoid(mm3(ne_b, wgn[...]) + bgn[...])
    msg_out[...] = core2 * gate2 * snw[...]


def _main_body_first(ai, aj, ef, sew, snw, wve, wge, wvn, wgn,
                     bve, bge, bvn, bgn, ne_out, msg_out):
    _main_body(None, ai, aj, ef, sew, snw, wve, wge, wvn, wgn,
               bve, bge, bvn, bgn, ne_out, msg_out)


def _tc_main(part, ne_acc, atom_i, atom_j, ef, sew, snw, wve, wge, wvn, wgn,
             bve, bge, bvn, bgn):
    epart = PART_CHUNKS[part] * CHUNK
    boff = PART_STARTS[part] * CHUNK // BE
    nbe = epart // BE
    first = part == 0
    pb = lambda i: (i, 0)
    gb = lambda i, boff=boff: (i + boff, 0)
    wb = lambda i: (0, 0)
    specs = [
        pl.BlockSpec((BE, D), pb),
        pl.BlockSpec((BE, D), pb),
        pl.BlockSpec((BE, D), gb),
        pl.BlockSpec((BE, D), gb),
        pl.BlockSpec((BE, D), gb),
        pl.BlockSpec((3 * D, D), wb),
        pl.BlockSpec((3 * D, D), wb),
        pl.BlockSpec((3 * D, D), wb),
        pl.BlockSpec((3 * D, D), wb),
        pl.BlockSpec((1, D), wb),
        pl.BlockSpec((1, D), wb),
        pl.BlockSpec((1, D), wb),
        pl.BlockSpec((1, D), wb),
    ]
    if not first:
        specs = [pl.BlockSpec(memory_space=pl.ANY)] + specs
    call = pl.pallas_call(
        _main_body_first if first else _main_body,
        grid=(nbe,),
        in_specs=specs,
        out_specs=[pl.BlockSpec((BE, D), gb), pl.BlockSpec((BE, D), pb)],
        out_shape=[jax.ShapeDtypeStruct((E, D), jnp.float32),
                   jax.ShapeDtypeStruct((epart, D), jnp.float32)],
        input_output_aliases={} if first else {0: 0},
    )
    args = (atom_i, atom_j, ef, sew, snw, wve, wge, wvn, wgn,
            bve, bge, bvn, bgn)
    if not first:
        args = (ne_acc,) + args
    return call(*args)


def _final_body(nf, a0, a1, wout, out):
    a = (a0[0] + a0[1]) + (a1[0] + a1[1])
    out[...] = nf[...] + jnp.dot(a, wout[...],
                                 preferred_element_type=jnp.float32)


def _tc_final(nf, aggs, wout):
    ab = lambda i: (0, i, 0)
    return pl.pallas_call(
        _final_body,
        grid=(N // BN,),
        in_specs=[pl.BlockSpec((BN, D), lambda i: (i, 0))]
        + [pl.BlockSpec((NC, BN, D), ab)] * len(aggs)
        + [pl.BlockSpec((D, D), lambda i: (0, 0))],
        out_specs=pl.BlockSpec((BN, D), lambda i: (i, 0)),
        out_shape=jax.ShapeDtypeStruct((N, D), jnp.float32),
    )(nf, *aggs, wout)


def kernel(node_features, edge_features, state_attr, shared_node_weights,
           shared_edge_weights, Wg_e, bg_e, Wv_e, bv_e,
           Wg_n, bg_n, Wv_n, bv_n, W_out, edge_index):
    ei_flat = edge_index.astype(jnp.int32).reshape(2 * E)
    dst3 = edge_index[1].astype(jnp.int32).reshape(E // CHUNK, 1, CHUNK)
    zeros = jnp.zeros((NPAD, D), jnp.float32)
    biases = (bv_e.reshape(1, D), bg_e.reshape(1, D),
              bv_n.reshape(1, D), bg_n.reshape(1, D))

    ne_full = None
    msgs = []
    for p in range(P):
        atom_i, atom_j = _make_gather(p)(node_features, ei_flat)
        ne_full, msg = _tc_main(
            p, ne_full, atom_i, atom_j, edge_features,
            shared_edge_weights, shared_node_weights,
            Wv_e, Wg_e, Wv_n, Wg_n, *biases)
        msgs.append(msg)

    agg_a = _make_scatter((0, 1, 2))(msgs[0], msgs[1], msgs[2], dst3, zeros)
    agg_b = _make_scatter((3, 4))(msgs[3], msgs[4], dst3, zeros)

    new_node = _tc_final(node_features, [agg_a, agg_b], W_out)
    return new_node, ne_full, state_attr


# speedup vs baseline: 1.0253x; 1.0181x over previous
import jax
import jax.numpy as jnp
from jax import lax
from jax.experimental import pallas as pl
from jax.experimental.pallas import tpu as pltpu
from jax.experimental.pallas import tpu_sc as plsc

N = 10000
E = 320000
D = 128

NC = 2
NS = 16
NW = NC * NS
CHUNK = 128
PART_CHUNKS = (500, 500, 500, 500, 500)
PART_STARTS = (0, 500, 1000, 1500, 2000)
P = len(PART_CHUNKS)
RING = 3
SRING = 2
NPAD = 10240
ROWS_PER_TILE = NPAD // NS
BE = 6400
BN = 2000

_MESH = plsc.VectorSubcoreMesh(core_axis_name="c", subcore_axis_name="s")


def _wid():
    return lax.axis_index("s") * NC + lax.axis_index("c")


def _make_gather(part):
    nch = PART_CHUNKS[part]
    pbase = PART_STARTS[part]
    cpw = nch // NW
    left = nch - cpw * NW
    rounds = cpw // RING
    rem = cpw - rounds * RING
    epart = nch * CHUNK

    def body(nf_hbm, ei_hbm, ai_hbm, aj_hbm,
             idx_s, idx_d, rs0, rs1, rs2, rd0, rd1, rd2,
             sg0, sg1, sg2, sw0, sw1, sw2):
        wid = _wid()
        gbase0 = (pbase + wid * cpw) * CHUNK
        lbase0 = wid * cpw * CHUNK
        rs = (rs0, rs1, rs2)
        rd = (rd0, rd1, rd2)
        sg = (sg0, sg1, sg2)
        sw = (sw0, sw1, sw2)

        pltpu.sync_copy(ei_hbm.at[pl.ds(gbase0, cpw * CHUNK)], idx_s)
        pltpu.sync_copy(ei_hbm.at[pl.ds(E + gbase0, cpw * CHUNK)], idx_d)

        def issue_gather(c, b):
            off = c * CHUNK
            pltpu.async_copy(nf_hbm.at[idx_s.at[pl.ds(off, CHUNK)]],
                             rs[b], sg[b])
            pltpu.async_copy(nf_hbm.at[idx_d.at[pl.ds(off, CHUNK)]],
                             rd[b], sg[b])

        def wait_gather(b):
            pltpu.make_async_copy(nf_hbm.at[pl.ds(0, CHUNK)],
                                  rs[b], sg[b]).wait()
            pltpu.make_async_copy(nf_hbm.at[pl.ds(0, CHUNK)],
                                  rd[b], sg[b]).wait()

        def issue_wb(c, b):
            base = lbase0 + c * CHUNK
            pltpu.async_copy(rs[b], ai_hbm.at[pl.ds(base, CHUNK)], sw[b])
            pltpu.async_copy(rd[b], aj_hbm.at[pl.ds(base, CHUNK)], sw[b])

        def wait_wb(b):
            pltpu.make_async_copy(rs[b], ai_hbm.at[pl.ds(0, CHUNK)],
                                  sw[b]).wait()
            pltpu.make_async_copy(rd[b], aj_hbm.at[pl.ds(0, CHUNK)],
                                  sw[b]).wait()

        for b in range(RING):
            issue_gather(b, b)

        def step(k, carry):
            for b in range(RING):
                c = k * RING + b
                wait_gather(b)
                issue_wb(c, b)
                wait_wb(b)

                @pl.when(k < rounds - 1)
                def _():
                    issue_gather(c + RING, b)
            return carry

        lax.fori_loop(0, rounds, step, 0)

        for c in range(rounds * RING, cpw):
            b = c % RING
            issue_gather(c, b)
            wait_gather(b)
            issue_wb(c, b)
            wait_wb(b)

        @pl.when(wid < left)
        def _():
            lc = NW * cpw + wid
            gb = (pbase + lc) * CHUNK
            lb = lc * CHUNK
            pltpu.sync_copy(ei_hbm.at[pl.ds(gb, CHUNK)],
                            idx_s.at[pl.ds(0, CHUNK)])
            pltpu.sync_copy(ei_hbm.at[pl.ds(E + gb, CHUNK)],
                            idx_d.at[pl.ds(0, CHUNK)])
            pltpu.async_copy(nf_hbm.at[idx_s.at[pl.ds(0, CHUNK)]], rs0, sg0)
            pltpu.async_copy(nf_hbm.at[idx_d.at[pl.ds(0, CHUNK)]], rd0, sg0)
            wait_gather(0)
            pltpu.sync_copy(rs0, ai_hbm.at[pl.ds(lb, CHUNK)])
            pltpu.sync_copy(rd0, aj_hbm.at[pl.ds(lb, CHUNK)])

    return pl.kernel(
        body,
        out_type=(jax.ShapeDtypeStruct((epart, D), jnp.float32),
                  jax.ShapeDtypeStruct((epart, D), jnp.float32)),
        mesh=_MESH,
        scratch_types=(
            [pltpu.VMEM((cpw * CHUNK,), jnp.int32)] * 2
            + [pltpu.VMEM((CHUNK, D), jnp.float32)] * 6
            + [pltpu.SemaphoreType.DMA] * 6
        ),
    )


def _make_scatter(parts):
    nmsg = len(parts)
    max_cpw = max(PART_CHUNKS[p] // NW for p in parts)

    def body(*refs):
        msgs = refs[:nmsg]
        dst3_hbm, zero_hbm, out_hbm = refs[nmsg:nmsg + 3]
        idx_v, m0, m1, agg_sh, sl0, sl1, ss0, ss1 = refs[nmsg + 3:]
        cid = lax.axis_index("c")
        sid = lax.axis_index("s")
        r0 = sid * ROWS_PER_TILE
        pltpu.sync_copy(zero_hbm.at[pl.ds(r0, ROWS_PER_TILE)],
                        agg_sh.at[pl.ds(r0, ROWS_PER_TILE)])
        plsc.subcore_barrier()

        wid = sid * NC + cid
        m = (m0, m1)
        sl = (sl0, sl1)
        ss = (ss0, ss1)

        def run_phase(msg_hbm, part):
            nch = PART_CHUNKS[part]
            pbase = PART_STARTS[part]
            cpw = nch // NW
            left = nch - cpw * NW
            nrounds = cpw // SRING

            pltpu.sync_copy(dst3_hbm.at[pl.ds(pbase + wid * cpw, cpw)],
                            idx_v.at[pl.ds(0, cpw)])

            def issue_load(c, b):
                base = (wid * cpw + c) * CHUNK
                pltpu.async_copy(msg_hbm.at[pl.ds(base, CHUNK)], m[b], sl[b])

            def wait_load(b):
                pltpu.make_async_copy(msg_hbm.at[pl.ds(0, CHUNK)],
                                      m[b], sl[b]).wait()

            def issue_scatter(c, b):
                pltpu.async_copy(m[b], agg_sh.at[idx_v.at[c, 0]], ss[b],
                                 add=True)

            def wait_scatter(b):
                pltpu.make_async_copy(m[b], agg_sh.at[pl.ds(0, CHUNK)],
                                      ss[b]).wait()

            for b in range(SRING):
                issue_load(b, b)

            def step(k, carry):
                for b in range(SRING):
                    c = k * SRING + b
                    wait_load(b)
                    issue_scatter(c, b)
                    wait_scatter(b)

                    @pl.when(k < nrounds - 1)
                    def _():
                        issue_load(c + SRING, b)
                return carry

            lax.fori_loop(0, nrounds, step, 0)

            for c in range(nrounds * SRING, cpw):
                pltpu.sync_copy(
                    msg_hbm.at[pl.ds((wid * cpw + c) * CHUNK, CHUNK)], m0)
                pltpu.sync_copy(m0, agg_sh.at[idx_v.at[c, 0]], add=True)

            @pl.when(wid < left)
            def _():
                lc = NW * cpw + wid
                pltpu.sync_copy(dst3_hbm.at[pl.ds(pbase + lc, 1)],
                                idx_v.at[pl.ds(0, 1)])
                pltpu.sync_copy(msg_hbm.at[pl.ds(lc * CHUNK, CHUNK)], m0)
                pltpu.sync_copy(m0, agg_sh.at[idx_v.at[0, 0]], add=True)

        for i, part in enumerate(parts):
            run_phase(msgs[i], part)

        plsc.subcore_barrier()
        last = N - (NS - 1) * ROWS_PER_TILE

        @pl.when(sid < NS - 1)
        def _():
            pltpu.sync_copy(agg_sh.at[pl.ds(r0, ROWS_PER_TILE)],
                            out_hbm.at[cid, pl.ds(r0, ROWS_PER_TILE)])

        @pl.when(sid == NS - 1)
        def _():
            pltpu.sync_copy(agg_sh.at[pl.ds(r0, last)],
                            out_hbm.at[cid, pl.ds(r0, last)])

    return pl.kernel(
        body,
        out_type=jax.ShapeDtypeStruct((NC, N, D), jnp.float32),
        mesh=_MESH,
        scratch_types=(
            [pltpu.VMEM((max_cpw, 1, CHUNK), jnp.int32)]
            + [pltpu.VMEM((CHUNK, D), jnp.float32)] * 2
            + [pltpu.VMEM_SHARED((NPAD, D), jnp.float32)]
            + [pltpu.SemaphoreType.DMA] * 4
        ),
    )


def _main_body(ne_in, ai, aj, ef, sew, snw, wve, wge, wvn, wgn,
               bve, bge, bvn, bgn, ne_out, msg_out):
    del ne_in
    ai_ = ai[...].astype(jnp.bfloat16)
    aj_ = aj[...].astype(jnp.bfloat16)
    ef_ = ef[...]
    ef_b = ef_.astype(jnp.bfloat16)

    def mm3(x2, w):
        wb = w.astype(jnp.bfloat16)
        return (jnp.dot(ai_, wb[0:D], preferred_element_type=jnp.float32)
                + jnp.dot(x2, wb[D:2 * D], preferred_element_type=jnp.float32)
                + jnp.dot(aj_, wb[2 * D:3 * D],
                          preferred_element_type=jnp.float32))

    core = jax.nn.silu(mm3(ef_b, wve[...]) + bve[...])
    gate = jax.nn.sigmoid(mm3(ef_b, wge[...]) + bge[...])
    ne = ef_ + core * gate * sew[...]
    ne_out[...] = ne
    ne_b = ne.astype(jnp.bfloat16)
    core2 = jax.nn.silu(mm3(ne_b, wvn[...]) + bvn[...])
    gate2 = jax.nn.sigmoid(mm3(ne_b, wgn[...]) + bgn[...])
    msg_out[...] = core2 * gate2 * snw[...]


def _main_body_first(ai, aj, ef, sew, snw, wve, wge, wvn, wgn,
                     bve, bge, bvn, bgn, ne_out, msg_out):
    _main_body(None, ai, aj, ef, sew, snw, wve, wge, wvn, wgn,
               bve, bge, bvn, bgn, ne_out, msg_out)


def _tc_main(part, ne_acc, atom_i, atom_j, ef, sew, snw, wve, wge, wvn, wgn,
             bve, bge, bvn, bgn):
    epart = PART_CHUNKS[part] * CHUNK
    boff = PART_STARTS[part] * CHUNK // BE
    nbe = epart // BE
    first = part == 0
    pb = lambda i: (i, 0)
    gb = lambda i, boff=boff: (i + boff, 0)
    wb = lambda i: (0, 0)
    specs = [
        pl.BlockSpec((BE, D), pb),
        pl.BlockSpec((BE, D), pb),
        pl.BlockSpec((BE, D), gb),
        pl.BlockSpec((BE, D), gb),
        pl.BlockSpec((BE, D), gb),
        pl.BlockSpec((3 * D, D), wb),
        pl.BlockSpec((3 * D, D), wb),
        pl.BlockSpec((3 * D, D), wb),
        pl.BlockSpec((3 * D, D), wb),
        pl.BlockSpec((1, D), wb),
        pl.BlockSpec((1, D), wb),
        pl.BlockSpec((1, D), wb),
        pl.BlockSpec((1, D), wb),
    ]
    if not first:
        specs = [pl.BlockSpec(memory_space=pl.ANY)] + specs
    call = pl.pallas_call(
        _main_body_first if first else _main_body,
        grid=(nbe,),
        in_specs=specs,
        out_specs=[pl.BlockSpec((BE, D), gb), pl.BlockSpec((BE, D), pb)],
        out_shape=[jax.ShapeDtypeStruct((E, D), jnp.float32),
                   jax.ShapeDtypeStruct((epart, D), jnp.float32)],
        input_output_aliases={} if first else {0: 0},
    )
    args = (atom_i, atom_j, ef, sew, snw, wve, wge, wvn, wgn,
            bve, bge, bvn, bgn)
    if not first:
        args = (ne_acc,) + args
    return call(*args)


def _final_body(nf, a0, a1, wout, out):
    a = (a0[0] + a0[1]) + (a1[0] + a1[1])
    out[...] = nf[...] + jnp.dot(a, wout[...],
                                 preferred_element_type=jnp.float32)


def _tc_final(nf, aggs, wout):
    ab = lambda i: (0, i, 0)
    return pl.pallas_call(
        _final_body,
        grid=(N // BN,),
        in_specs=[pl.BlockSpec((BN, D), lambda i: (i, 0))]
        + [pl.BlockSpec((NC, BN, D), ab)] * len(aggs)
        + [pl.BlockSpec((D, D), lambda i: (0, 0))],
        out_specs=pl.BlockSpec((BN, D), lambda i: (i, 0)),
        out_shape=jax.ShapeDtypeStruct((N, D), jnp.float32),
    )(nf, *aggs, wout)


def kernel(node_features, edge_features, state_attr, shared_node_weights,
           shared_edge_weights, Wg_e, bg_e, Wv_e, bv_e,
           Wg_n, bg_n, Wv_n, bv_n, W_out, edge_index):
    ei_flat = edge_index.astype(jnp.int32).reshape(2 * E)
    dst3 = edge_index[1].astype(jnp.int32).reshape(E // CHUNK, 1, CHUNK)
    zeros = jnp.zeros((NPAD, D), jnp.float32)
    biases = (bv_e.reshape(1, D), bg_e.reshape(1, D),
              bv_n.reshape(1, D), bg_n.reshape(1, D))

    ne_full = None
    msgs = []
    for p in range(P):
        atom_i, atom_j = _make_gather(p)(node_features, ei_flat)
        ne_full, msg = _tc_main(
            p, ne_full, atom_i, atom_j, edge_features,
            shared_edge_weights, shared_node_weights,
            Wv_e, Wg_e, Wv_n, Wg_n, *biases)
        msgs.append(msg)

    agg_a = _make_scatter((0, 1, 2))(msgs[0], msgs[1], msgs[2], dst3, zeros)
    agg_b = _make_scatter((3, 4))(msgs[3], msgs[4], dst3, zeros)

    new_node = _tc_final(node_features, [agg_a, agg_b], W_out)
    return new_node, ne_full, state_attr
